# speedup vs baseline: 11.9743x; 11.9743x over previous
import numpy as np
import jax
import jax.numpy as jnp
from jax.experimental import pallas as pl
from jax.experimental.pallas import tpu as pltpu

SR = 16000
N_FFT = 1024
HOP = 256
N_MELS = 80
EPS = 1e-8
DRY_W, RIR_W = 3.0, 1.0
TIME_W, FREQ_W, MEL_W = 1.0, 0.5, 0.3
CONSIST_W, RIR_REG_W = 0.2, 0.1

T_LEN = 64000
N_FRAMES = 1 + T_LEN // HOP
N_FREQS = N_FFT // 2 + 1
FPAD = 640
MPAD = 128
ROWS_A = 264

FN1, FN2 = 512, 256
FN = FN1 * FN2
ROWS_IN = T_LEN // FN2


def _np_consts():
    f = {}
    n = np.arange(N_FFT, dtype=np.float64)
    win = 0.5 - 0.5 * np.cos(2.0 * np.pi * n / N_FFT)
    norm = 1.0 / np.sqrt(np.sum(win ** 2))
    k = np.arange(N_FREQS, dtype=np.float64)
    ang = 2.0 * np.pi * np.outer(n, k) / N_FFT
    C2 = np.zeros((N_FFT, FPAD))
    S2 = np.zeros((N_FFT, FPAD))
    C2[:, :N_FREQS] = win[:, None] * np.cos(ang) * norm
    S2[:, :N_FREQS] = win[:, None] * (-np.sin(ang)) * norm
    f["C2"], f["S2"] = C2, S2
    all_freqs = np.linspace(0.0, SR / 2.0, N_FREQS)
    hz2mel = lambda x: 2595.0 * np.log10(1.0 + x / 700.0)
    mel2hz = lambda m: 700.0 * (10.0 ** (m / 2595.0) - 1.0)
    m_pts = np.linspace(hz2mel(0.0), hz2mel(SR / 2.0), N_MELS + 2)
    f_pts = mel2hz(m_pts)
    f_diff = f_pts[1:] - f_pts[:-1]
    slopes = f_pts[None, :] - all_freqs[:, None]
    down = -slopes[:, :-2] / f_diff[:-1]
    up = slopes[:, 2:] / f_diff[1:]
    fb = np.maximum(0.0, np.minimum(down, up))
    FB = np.zeros((FPAD, MPAD))
    FB[:N_FREQS, :N_MELS] = fb
    f["FB"] = FB
    k1 = np.arange(FN1, dtype=np.float64)
    n2 = np.arange(FN2, dtype=np.float64)
    n1c = np.arange(FN2, dtype=np.float64)
    a1 = 2.0 * np.pi * np.outer(k1, n1c) / FN1
    W1r, W1i = np.cos(a1), -np.sin(a1)
    W1r[:, ROWS_IN:] = 0.0
    W1i[:, ROWS_IN:] = 0.0
    at = 2.0 * np.pi * np.outer(k1, n2) / FN
    f["Twr"], f["Twi"] = np.cos(at), -np.sin(at)
    a2 = 2.0 * np.pi * np.outer(n2, n2) / FN2
    f["W2r"], f["W2i"] = np.cos(a2), -np.sin(a2)
    f["V2r"], f["V2i"] = np.cos(a2), np.sin(a2)
    av = 2.0 * np.pi * np.outer(n2, k1) / FN1
    f["V1c"], f["V1s"] = np.cos(av) / FN, np.sin(av) / FN
    f["W1r"], f["W1i"] = W1r, W1i
    return {kk: np.asarray(vv, np.float32) for kk, vv in f.items()}


_C = _np_consts()


def _stft_body(pdf_ref, tdf_ref, c_ref, s_ref, fb_ref, out_ref):
    Xp = pdf_ref[0]
    Xt = tdf_ref[0]
    fp = jnp.concatenate([Xp[0:256], Xp[1:257], Xp[2:258], Xp[3:259]], axis=1)
    ft = jnp.concatenate([Xt[0:256], Xt[1:257], Xt[2:258], Xt[3:259]], axis=1)
    C = c_ref[...]
    S = s_ref[...]
    ReP = jnp.dot(fp, C, preferred_element_type=jnp.float32)
    ImP = jnp.dot(fp, S, preferred_element_type=jnp.float32)
    ReT = jnp.dot(ft, C, preferred_element_type=jnp.float32)
    ImT = jnp.dot(ft, S, preferred_element_type=jnp.float32)
    magP2 = ReP * ReP + ImP * ImP
    magT2 = ReT * ReT + ImT * ImT
    magP = jnp.sqrt(magP2)
    magT = jnp.sqrt(magT2)
    rows = jax.lax.broadcasted_iota(jnp.int32, (256, FPAD), 0)
    fmask = rows < N_FRAMES
    magdiff = jnp.where(fmask, jnp.abs(magP - magT), 0.0)
    num = ReP * ReT + ImP * ImT
    den = magP * magT + 1e-30
    cosd = jnp.where(fmask, num / den, 0.0)
    FBm = fb_ref[...]
    melP = jnp.dot(magP2, FBm, preferred_element_type=jnp.float32)
    melT = jnp.dot(magT2, FBm, preferred_element_type=jnp.float32)
    rows_m = jax.lax.broadcasted_iota(jnp.int32, (256, MPAD), 0)
    meldiff = jnp.where(rows_m < N_FRAMES,
                        jnp.abs(jnp.log(melP + EPS) - jnp.log(melT + EPS)), 0.0)
    rows_t = jax.lax.broadcasted_iota(jnp.int32, (ROWS_A, 256), 0)
    tmask = (rows_t >= 2) & (rows_t < 252)
    tdiff = jnp.where(tmask, jnp.abs(Xp - Xt), 0.0)
    r0 = jnp.sum(magdiff, axis=0, keepdims=True)
    r1 = jnp.sum(cosd, axis=0, keepdims=True)
    r2 = jnp.sum(meldiff, axis=0, keepdims=True)
    r2 = jnp.concatenate([r2, jnp.zeros((1, FPAD - MPAD), jnp.float32)], axis=1)
    r3 = jnp.sum(tdiff, axis=0, keepdims=True)
    r3 = jnp.concatenate([r3, jnp.zeros((1, FPAD - 256), jnp.float32)], axis=1)
    out_ref[0] = jnp.concatenate(
        [r0, r1, r2, r3, jnp.zeros((4, FPAD), jnp.float32)], axis=0)


def _fft_fwd(X, w1r, w1i, twr, twi, w2r, w2i):
    Ar = jnp.dot(w1r, X, preferred_element_type=jnp.float32)
    Ai = jnp.dot(w1i, X, preferred_element_type=jnp.float32)
    Atr = Ar * twr - Ai * twi
    Ati = Ar * twi + Ai * twr
    Fr = (jnp.dot(Atr, w2r, preferred_element_type=jnp.float32)
          - jnp.dot(Ati, w2i, preferred_element_type=jnp.float32))
    Fi = (jnp.dot(Atr, w2i, preferred_element_type=jnp.float32)
          + jnp.dot(Ati, w2r, preferred_element_type=jnp.float32))
    return Fr, Fi


def _conv_body(xd_ref, xr_ref, xtr_ref, xmix_ref,
               w1r_ref, w1i_ref, twr_ref, twi_ref, w2r_ref, w2i_ref,
               v2r_ref, v2i_ref, v1c_ref, v1s_ref, out_ref):
    Xd = xd_ref[0]
    Xr = xr_ref[0]
    Xtr = xtr_ref[0]
    Xmx = xmix_ref[0]
    w1r, w1i = w1r_ref[...], w1i_ref[...]
    twr, twi = twr_ref[...], twi_ref[...]
    w2r, w2i = w2r_ref[...], w2i_ref[...]
    Dr, Di = _fft_fwd(Xd, w1r, w1i, twr, twi, w2r, w2i)
    Rr, Ri = _fft_fwd(Xr, w1r, w1i, twr, twi, w2r, w2i)
    Pr = Dr * Rr - Di * Ri
    Pi = Dr * Ri + Di * Rr
    v2r, v2i = v2r_ref[...], v2i_ref[...]
    Br = (jnp.dot(Pr, v2r, preferred_element_type=jnp.float32)
          - jnp.dot(Pi, v2i, preferred_element_type=jnp.float32))
    Bi = (jnp.dot(Pr, v2i, preferred_element_type=jnp.float32)
          + jnp.dot(Pi, v2r, preferred_element_type=jnp.float32))
    B2r = Br * twr + Bi * twi
    B2i = Bi * twr - Br * twi
    y = (jnp.dot(v1c_ref[...], B2r, preferred_element_type=jnp.float32)
         - jnp.dot(v1s_ref[...], B2i, preferred_element_type=jnp.float32))
    rows = jax.lax.broadcasted_iota(jnp.int32, (FN2, FN2), 0)
    lanes = jax.lax.broadcasted_iota(jnp.int32, (FN2, FN2), 1)
    valid = rows < ROWS_IN
    cdiff = jnp.where(valid, jnp.abs(y - Xmx), 0.0)
    rtdiff = jnp.abs(Xr - Xtr)
    rabs = jnp.abs(Xr)
    rsq = Xr * Xr
    t = rows * FN2 + lanes
    seg = T_LEN // 4
    q = []
    for i in range(4):
        m = (t >= i * seg) & (t < (i + 1) * seg)
        q.append(jnp.sum(jnp.where(m, rsq, 0.0), axis=0, keepdims=True))
    r0 = jnp.sum(cdiff, axis=0, keepdims=True)
    r1 = jnp.sum(rtdiff, axis=0, keepdims=True)
    r2 = jnp.sum(rabs, axis=0, keepdims=True)
    out_ref[0] = jnp.concatenate(
        [r0, r1, r2, q[0], q[1], q[2], q[3],
         jnp.zeros((1, FN2), jnp.float32)], axis=0)


def _full_spec(shape):
    nd = len(shape)
    return pl.BlockSpec(shape, lambda s, _nd=nd: (0,) * _nd)


@jax.jit
def kernel(pred_dry, pred_rir, target_dry, target_rir, mix):
    B = pred_dry.shape[0] * pred_dry.shape[1]
    pd = pred_dry.reshape(B, T_LEN)
    td = target_dry.reshape(B, T_LEN)
    pr = pred_rir.reshape(B, T_LEN)
    tr = target_rir.reshape(B, T_LEN)
    mx = mix.reshape(B, T_LEN)

    def frame_prep(x):
        xp = jnp.pad(x, ((0, 0), (N_FFT // 2, N_FFT // 2)), mode="reflect")
        xp = jnp.pad(xp, ((0, 0), (0, ROWS_A * 256 - xp.shape[1])))
        return xp.reshape(B, ROWS_A, 256)

    pdf = frame_prep(pd)
    tdf = frame_prep(td)
    c2 = jnp.asarray(_C["C2"])
    s2 = jnp.asarray(_C["S2"])
    fbm = jnp.asarray(_C["FB"])

    outA = pl.pallas_call(
        _stft_body,
        grid=(B,),
        in_specs=[
            pl.BlockSpec((1, ROWS_A, 256), lambda s: (s, 0, 0)),
            pl.BlockSpec((1, ROWS_A, 256), lambda s: (s, 0, 0)),
            _full_spec((N_FFT, FPAD)),
            _full_spec((N_FFT, FPAD)),
            _full_spec((FPAD, MPAD)),
        ],
        out_specs=pl.BlockSpec((1, 8, FPAD), lambda s: (s, 0, 0)),
        out_shape=jax.ShapeDtypeStruct((B, 8, FPAD), jnp.float32),
        compiler_params=pltpu.CompilerParams(
            dimension_semantics=("parallel",),
            vmem_limit_bytes=100 * 1024 * 1024,
        ),
    )(pdf, tdf, c2, s2, fbm)

    def conv_prep(x):
        return jnp.pad(x.reshape(B, ROWS_IN, FN2),
                       ((0, 0), (0, FN2 - ROWS_IN), (0, 0)))

    consts = [jnp.asarray(_C[k]) for k in
              ("W1r", "W1i", "Twr", "Twi", "W2r", "W2i",
               "V2r", "V2i", "V1c", "V1s")]
    const_specs = [_full_spec(c.shape) for c in consts]

    outB = pl.pallas_call(
        _conv_body,
        grid=(B,),
        in_specs=[pl.BlockSpec((1, FN2, FN2), lambda s: (s, 0, 0))] * 4
        + const_specs,
        out_specs=pl.BlockSpec((1, 8, FN2), lambda s: (s, 0, 0)),
        out_shape=jax.ShapeDtypeStruct((B, 8, FN2), jnp.float32),
        compiler_params=pltpu.CompilerParams(
            dimension_semantics=("parallel",),
            vmem_limit_bytes=100 * 1024 * 1024,
        ),
    )(conv_prep(pd), conv_prep(pr), conv_prep(tr), conv_prep(mx), *consts)

    sA = jnp.sum(outA, axis=(0, 2))
    sB = jnp.sum(outB, axis=(0, 2))

    nfb = B * N_FREQS * N_FRAMES
    n_t = B * T_LEN
    mag_l1 = sA[0] / nfb
    phase = 1.0 - sA[1] / nfb
    freq_loss = mag_l1 + 0.1 * phase
    mel_loss = sA[2] / (B * N_MELS * N_FRAMES)
    dry_time = sA[3] / n_t
    total_dry = TIME_W * dry_time + FREQ_W * freq_loss + MEL_W * mel_loss

    consist = sB[0] / n_t
    rir_time = sB[1] / n_t
    sparsity = sB[2] / n_t
    e = sB[3:7] / (B * (T_LEN // 4))
    decay = (jax.nn.relu(e[1] - 0.8 * e[0]) + jax.nn.relu(e[2] - 0.8 * e[1])
             + jax.nn.relu(e[3] - 0.8 * e[2]))
    rir_reg = sparsity + decay
    total_rir = rir_time + RIR_REG_W * rir_reg

    total = DRY_W * total_dry + RIR_W * total_rir + CONSIST_W * consist
    return total.astype(jnp.float32)
